# baseline (speedup 1.0000x reference)
import jax
import jax.numpy as jnp
from jax import lax
from jax.experimental import pallas as pl
from jax.experimental.pallas import tpu as pltpu
from jax.experimental.pallas import tpu_sc as plsc

N = 16384
D = 4096
L = 16
NUM_WORKERS = 32
ROWS_PER_TILE = N // NUM_WORKERS
R = 4
RO = 2
CHUNKS = ROWS_PER_TILE // R
OCHUNKS = ROWS_PER_TILE // RO
NBUF_IN = 4
NBUF_OUT = 4
GROUPS = CHUNKS // NBUF_IN


def _body(x_hbm, perm_hbm, out_hbm, perm_v,
          in0, in1, in2, in3, out0, out1, out2, out3,
          sin0, sin1, sin2, sin3, sout0, sout1, sout2, sout3,
          sperm):
    num_cores = 2
    wid = lax.axis_index("s") * num_cores + lax.axis_index("c")
    row0 = wid * ROWS_PER_TILE

    def in_copy(g, buf, sem):
        return pltpu.make_async_copy(
            x_hbm.at[pl.ds(row0 + g * R, R), :], buf, sem)

    def out_copy(h, buf, sem):
        return pltpu.make_async_copy(
            buf, out_hbm.at[pl.ds(row0 + h * RO, RO), :], sem)

    perm_copy = pltpu.make_async_copy(perm_hbm, perm_v, sperm)

    in_slots = ((in0, sin0), (in1, sin1), (in2, sin2), (in3, sin3))
    out_slots = ((out0, sout0), (out1, sout1), (out2, sout2), (out3, sout3))

    for b, (inb, sin) in enumerate(in_slots):
        in_copy(b, inb, sin).start()
    perm_copy.start()
    perm_copy.wait()

    def compute(in_b, out_b, half):
        @plsc.parallel_loop(0, D // L, unroll=8)
        def _col(j):
            idx = perm_v[pl.ds(j * L, L)]
            for r in range(RO):
                row_ids = jnp.full((L,), half * RO + r, jnp.int32)
                out_b[r, pl.ds(j * L, L)] = plsc.load_gather(
                    in_b, [row_ids, idx])

    @pl.loop(0, GROUPS)
    def _iter(i):
        for b, (inb, sin) in enumerate(in_slots):
            g = i * NBUF_IN + b
            in_copy(g, inb, sin).wait()

            for half in range(2):
                h = g * 2 + half
                outb, sout = out_slots[(b * 2 + half) % NBUF_OUT]

                @pl.when(h >= NBUF_OUT)
                def _():
                    out_copy(h - NBUF_OUT, outb, sout).wait()

                compute(inb, outb, half)
                out_copy(h, outb, sout).start()

            @pl.when(i < GROUPS - 1)
            def _():
                in_copy(g + NBUF_IN, inb, sin).start()

    for k in range(NBUF_OUT):
        h = OCHUNKS - NBUF_OUT + k
        outb, sout = out_slots[h % NBUF_OUT]
        out_copy(h, outb, sout).wait()


def kernel(x, perm):
    mesh = plsc.VectorSubcoreMesh(core_axis_name="c", subcore_axis_name="s")
    f = pl.kernel(
        _body,
        out_type=jax.ShapeDtypeStruct((N, D), jnp.float32),
        mesh=mesh,
        compiler_params=pltpu.CompilerParams(
            use_tc_tiling_on_sc=True, needs_layout_passes=False),
        scratch_types=[
            pltpu.VMEM((D,), jnp.int32),
            pltpu.VMEM((R, D), jnp.float32),
            pltpu.VMEM((R, D), jnp.float32),
            pltpu.VMEM((R, D), jnp.float32),
            pltpu.VMEM((R, D), jnp.float32),
            pltpu.VMEM((RO, D), jnp.float32),
            pltpu.VMEM((RO, D), jnp.float32),
            pltpu.VMEM((RO, D), jnp.float32),
            pltpu.VMEM((RO, D), jnp.float32),
            pltpu.SemaphoreType.DMA,
            pltpu.SemaphoreType.DMA,
            pltpu.SemaphoreType.DMA,
            pltpu.SemaphoreType.DMA,
            pltpu.SemaphoreType.DMA,
            pltpu.SemaphoreType.DMA,
            pltpu.SemaphoreType.DMA,
            pltpu.SemaphoreType.DMA,
            pltpu.SemaphoreType.DMA,
        ],
    )
    return f(x, perm.astype(jnp.int32))


# speedup vs baseline: 1.0044x; 1.0044x over previous
import jax
import jax.numpy as jnp
from jax import lax
from jax.experimental import pallas as pl
from jax.experimental.pallas import tpu as pltpu
from jax.experimental.pallas import tpu_sc as plsc

N = 16384
D = 4096
L = 16
NUM_WORKERS = 32
ROWS_PER_TILE = N // NUM_WORKERS
R = 4
CHUNKS = ROWS_PER_TILE // R
NBUF_IN = 4
NBUF_OUT = 2
GROUPS = CHUNKS // NBUF_IN


def _body(x_hbm, perm_hbm, out_hbm, perm_v,
          in0, in1, in2, in3, out0, out1,
          sin0, sin1, sin2, sin3, sout0, sout1):
    num_cores = 2
    wid = lax.axis_index("s") * num_cores + lax.axis_index("c")
    row0 = wid * ROWS_PER_TILE

    pltpu.sync_copy(perm_hbm, perm_v)

    def in_copy(g, buf, sem):
        return pltpu.make_async_copy(
            x_hbm.at[pl.ds(row0 + g * R, R), :], buf, sem)

    def out_copy(g, buf, sem):
        return pltpu.make_async_copy(
            buf, out_hbm.at[pl.ds(row0 + g * R, R), :], sem)

    def compute(in_b, out_b):
        @plsc.parallel_loop(0, D // L, unroll=8)
        def _col(j):
            idx = perm_v[pl.ds(j * L, L)]
            for r in range(R):
                row_ids = jnp.full((L,), r, jnp.int32)
                out_b[r, pl.ds(j * L, L)] = plsc.load_gather(
                    in_b, [row_ids, idx])

    in_slots = ((in0, sin0), (in1, sin1), (in2, sin2), (in3, sin3))
    out_slots = ((out0, sout0), (out1, sout1))

    for b, (inb, sin) in enumerate(in_slots):
        in_copy(b, inb, sin).start()

    @pl.loop(0, GROUPS)
    def _iter(i):
        for b, (inb, sin) in enumerate(in_slots):
            outb, sout = out_slots[b % NBUF_OUT]
            g = i * NBUF_IN + b
            in_copy(g, inb, sin).wait()

            @pl.when(g >= NBUF_OUT)
            def _():
                out_copy(g - NBUF_OUT, outb, sout).wait()

            compute(inb, outb)
            out_copy(g, outb, sout).start()

            @pl.when(i < GROUPS - 1)
            def _():
                in_copy(g + NBUF_IN, inb, sin).start()

    out_copy(CHUNKS - 2, out0, sout0).wait()
    out_copy(CHUNKS - 1, out1, sout1).wait()


def kernel(x, perm):
    mesh = plsc.VectorSubcoreMesh(core_axis_name="c", subcore_axis_name="s")
    f = pl.kernel(
        _body,
        out_type=jax.ShapeDtypeStruct((N, D), jnp.float32),
        mesh=mesh,
        compiler_params=pltpu.CompilerParams(
            use_tc_tiling_on_sc=True, needs_layout_passes=False),
        scratch_types=[
            pltpu.VMEM((D,), jnp.int32),
            pltpu.VMEM((R, D), jnp.float32),
            pltpu.VMEM((R, D), jnp.float32),
            pltpu.VMEM((R, D), jnp.float32),
            pltpu.VMEM((R, D), jnp.float32),
            pltpu.VMEM((R, D), jnp.float32),
            pltpu.VMEM((R, D), jnp.float32),
            pltpu.SemaphoreType.DMA,
            pltpu.SemaphoreType.DMA,
            pltpu.SemaphoreType.DMA,
            pltpu.SemaphoreType.DMA,
            pltpu.SemaphoreType.DMA,
            pltpu.SemaphoreType.DMA,
        ],
    )
    return f(x, perm.astype(jnp.int32))
